# Initial kernel scaffold; baseline (speedup 1.0000x reference)
import jax
import jax.numpy as jnp
from jax.experimental import pallas as pl


def kernel(feat, edge_index, etypes, coeff, W, h_bias, loop_weight):
    raise NotImplementedError("write your pallas kernel here")


# speedup vs baseline: 94.9830x; 94.9830x over previous
import functools

import jax
import jax.numpy as jnp
from jax import lax
from jax.experimental import pallas as pl
from jax.experimental.pallas import tpu as pltpu
from jax.experimental.pallas import tpu_sc as plsc

N_NODES = 10000
N_EDGES = 320000
D = 128
NUM_RELS = 16
NUM_BASES = 2

NW = 32
BATCH = 128
NBLK = 79
E_PAD = NW * NBLK * BATCH
NACC = 10240
ROWS_PER_TILE = NACC // 16
TRASH_ROW = N_NODES
BLK = 2000
NRB = N_NODES // BLK


def _expand_body(coeff_ref, feat_ref, w_ref, lw_ref, b_ref, out_ref):
    r = pl.program_id(0)
    f = feat_ref[...]

    @pl.when(r < NUM_RELS)
    def _():
        wr = coeff_ref[r, 0] * w_ref[0]
        for b in range(1, NUM_BASES):
            wr += coeff_ref[r, b] * w_ref[b]
        out_ref[...] = jnp.dot(f, wr, preferred_element_type=jnp.float32)

    @pl.when(r == NUM_RELS)
    def _():
        out_ref[...] = (
            jnp.dot(f, w_ref[NUM_BASES] + lw_ref[...],
                    preferred_element_type=jnp.float32)
            + b_ref[...]
        )


def _expand(feat, coeff, w, lw, bias):
    return pl.pallas_call(
        _expand_body,
        grid=(NUM_RELS + 1, NRB),
        in_specs=[
            pl.BlockSpec(memory_space=pltpu.SMEM),
            pl.BlockSpec((BLK, D), lambda r, n: (n, 0)),
            pl.BlockSpec((NUM_BASES + 1, D, D), lambda r, n: (0, 0, 0)),
            pl.BlockSpec((D, D), lambda r, n: (0, 0)),
            pl.BlockSpec((1, D), lambda r, n: (0, 0)),
        ],
        out_specs=pl.BlockSpec((BLK, D), lambda r, n: (r * NRB + n, 0)),
        out_shape=jax.ShapeDtypeStruct(((NUM_RELS + 1) * N_NODES, D),
                                       jnp.float32),
    )(coeff, feat, w, lw, bias)


@functools.partial(
    pl.kernel,
    out_type=jax.ShapeDtypeStruct((2, NACC, D), jnp.float32),
    mesh=plsc.VectorSubcoreMesh(core_axis_name="c", subcore_axis_name="s"),
    scratch_types=[
        pltpu.VMEM((BATCH,), jnp.int32),
        pltpu.VMEM((BATCH,), jnp.int32),
        pltpu.VMEM((BATCH,), jnp.int32),
        pltpu.VMEM((BATCH,), jnp.int32),
        pltpu.VMEM((BATCH, D), jnp.float32),
        pltpu.VMEM_SHARED((NACC, D), jnp.float32),
        pltpu.SemaphoreType.DMA,
    ],
)
def _sc_edges(xflat, srcp, dstp, etp, zrows, out,
              src_v, dst_v, et_v, gidx_v, rows_v, acc, sem):
    i32 = jnp.int32
    c = lax.axis_index("c").astype(i32)
    s = lax.axis_index("s").astype(i32)
    wid = s * i32(2) + c
    tile_row0 = s * i32(ROWS_PER_TILE)

    for k in range(ROWS_PER_TILE // BATCH):
        pltpu.sync_copy(zrows, acc.at[pl.ds(tile_row0 + i32(k * BATCH), BATCH)])
    plsc.subcore_barrier()

    def body(b, carry):
        off = wid * i32(NBLK * BATCH) + b * i32(BATCH)
        pltpu.sync_copy(srcp.at[pl.ds(off, BATCH)], src_v)
        pltpu.sync_copy(dstp.at[pl.ds(off, BATCH)], dst_v)
        pltpu.sync_copy(etp.at[pl.ds(off, BATCH)], et_v)
        for j in range(BATCH // 16):
            sl = pl.ds(j * 16, 16)
            gidx_v[sl] = et_v[sl] * i32(N_NODES) + src_v[sl]
        pltpu.async_copy(xflat.at[gidx_v], rows_v, sem).wait()
        pltpu.sync_copy(rows_v, acc.at[dst_v], add=True)
        return carry

    lax.fori_loop(i32(0), i32(NBLK), body, i32(0))
    plsc.subcore_barrier()
    pltpu.sync_copy(acc.at[pl.ds(tile_row0, ROWS_PER_TILE)],
                    out.at[c, pl.ds(tile_row0, ROWS_PER_TILE)])


def _final_body(p0_ref, p1_ref, s_ref, out_ref):
    out_ref[...] = p0_ref[0] + p1_ref[0] + s_ref[...]


def _final(partials, xflat):
    return pl.pallas_call(
        _final_body,
        grid=(NRB,),
        in_specs=[
            pl.BlockSpec((1, BLK, D), lambda n: (0, n, 0)),
            pl.BlockSpec((1, BLK, D), lambda n: (1, n, 0)),
            pl.BlockSpec((BLK, D), lambda n: (NUM_RELS * NRB + n, 0)),
        ],
        out_specs=pl.BlockSpec((BLK, D), lambda n: (n, 0)),
        out_shape=jax.ShapeDtypeStruct((N_NODES, D), jnp.float32),
    )(partials, partials, xflat)


def kernel(feat, edge_index, etypes, coeff, W, h_bias, loop_weight):
    feat = feat.astype(jnp.float32)
    src = edge_index[0].astype(jnp.int32)
    dst = edge_index[1].astype(jnp.int32)
    et = etypes.astype(jnp.int32)

    with jax.enable_x64(False):
        pad = E_PAD - N_EDGES
        src_p = jnp.concatenate([src, jnp.zeros((pad,), jnp.int32)])
        dst_p = jnp.concatenate([dst, jnp.full((pad,), TRASH_ROW, jnp.int32)])
        et_p = jnp.concatenate([et, jnp.zeros((pad,), jnp.int32)])

        xflat = _expand(feat, coeff.astype(jnp.float32),
                        W.astype(jnp.float32),
                        loop_weight.astype(jnp.float32),
                        h_bias.astype(jnp.float32).reshape(1, D))
        zrows = jnp.zeros((BATCH, D), jnp.float32)
        partials = _sc_edges(xflat, src_p, dst_p, et_p, zrows)
        out = _final(partials, xflat)
    return out.astype(jnp.float64)
